# Initial kernel scaffold; baseline (speedup 1.0000x reference)
import functools

import jax
import jax.numpy as jnp
from jax import lax
from jax.experimental import pallas as pl
from jax.experimental.pallas import tpu as pltpu
from jax.experimental.pallas import tpu_sc as plsc

N = 10000
E = 320000
D = 128
DH = 64
NC = 2
NS = 16
LANE = 128
CH = 158
EPAD = NS * CH * LANE - E
NPT = N // NS
NPAD = 10240
DUMMY = N

_vmesh = plsc.VectorSubcoreMesh(core_axis_name="c", subcore_axis_name="s")


@functools.partial(
    pl.kernel,
    out_type=jax.ShapeDtypeStruct((NC, NPAD), jnp.float32),
    mesh=_vmesh,
    scratch_types=[
        pltpu.VMEM((CH // 2, LANE), jnp.int32),
        pltpu.VMEM((LANE,), jnp.float32),
        pltpu.VMEM((NPAD // NS,), jnp.float32),
        pltpu.VMEM_SHARED((NPAD,), jnp.float32),
    ],
)
def _deg_kernel(dst_hbm, deg_hbm, dst_v, ones_v, zeros_v, deg_sh):
    c = lax.axis_index("c")
    s = lax.axis_index("s")

    def fill(i, _):
        zeros_v[pl.ds(i * 16, 16)] = jnp.zeros((16,), jnp.float32)
        return _

    lax.fori_loop(0, (NPAD // NS) // 16, fill, 0)

    def fill1(i, _):
        ones_v[pl.ds(i * 16, 16)] = jnp.ones((16,), jnp.float32)
        return _

    lax.fori_loop(0, LANE // 16, fill1, 0)

    base = s * (NPAD // NS)
    pltpu.sync_copy(zeros_v, deg_sh.at[pl.ds(base, NPAD // NS)])
    pltpu.sync_copy(dst_hbm.at[s].at[pl.ds(c * (CH // 2), CH // 2)], dst_v)
    plsc.subcore_barrier()

    def chunk(j, _):
        pltpu.sync_copy(ones_v, deg_sh.at[dst_v.at[j]], add=True)
        return _

    lax.fori_loop(0, CH // 2, chunk, 0)
    plsc.subcore_barrier()
    pltpu.sync_copy(deg_sh.at[pl.ds(base, NPAD // NS)],
                    deg_hbm.at[c].at[pl.ds(base, NPAD // NS)])


def _mm_body(x_ref, w_ref, deg0_ref, deg1_ref, hp_ref, dis_ref):
    deg = deg0_ref[...] + deg1_ref[...] + 1.0
    dis = lax.rsqrt(deg)
    h = jnp.dot(x_ref[...], w_ref[...], preferred_element_type=jnp.float32)
    hp = h * dis[:, None]
    hp_ref[0] = hp[:, :DH]
    hp_ref[1] = hp[:, DH:]
    dis_ref[...] = dis


def _mm_call(x, W, deg0, deg1):
    blk = 1000
    grid = N // blk
    return pl.pallas_call(
        _mm_body,
        grid=(grid,),
        in_specs=[
            pl.BlockSpec((blk, D), lambda i: (i, 0)),
            pl.BlockSpec((D, D), lambda i: (0, 0)),
            pl.BlockSpec((blk,), lambda i: (i,)),
            pl.BlockSpec((blk,), lambda i: (i,)),
        ],
        out_specs=[
            pl.BlockSpec((NC, blk, DH), lambda i: (0, i, 0)),
            pl.BlockSpec((blk,), lambda i: (i,)),
        ],
        out_shape=[
            jax.ShapeDtypeStruct((NC, N, DH), jnp.float32),
            jax.ShapeDtypeStruct((N,), jnp.float32),
        ],
    )(x, W, deg0, deg1)


@functools.partial(
    pl.kernel,
    out_type=jax.ShapeDtypeStruct((NC, N, DH), jnp.float32),
    mesh=_vmesh,
    scratch_types=[
        pltpu.VMEM((CH, LANE), jnp.int32),
        pltpu.VMEM((CH, LANE), jnp.int32),
        pltpu.VMEM((LANE, DH), jnp.float32),
        pltpu.VMEM_SHARED((N + 16, DH), jnp.float32),
        pltpu.SemaphoreType.DMA,
    ],
)
def _scatter_kernel(hp_hbm, src_hbm, dst_hbm, acc_hbm, src_v, dst_v, rows_v,
                    acc_sh, sem):
    c = lax.axis_index("c")
    s = lax.axis_index("s")
    pltpu.sync_copy(src_hbm.at[s], src_v)
    pltpu.sync_copy(dst_hbm.at[s], dst_v)
    base = s * NPT
    pltpu.sync_copy(hp_hbm.at[c].at[pl.ds(base, NPT)],
                    acc_sh.at[pl.ds(base, NPT)])
    plsc.subcore_barrier()

    def chunk(j, _):
        pltpu.async_copy(hp_hbm.at[c].at[src_v.at[j]], rows_v, sem).wait()
        pltpu.sync_copy(rows_v, acc_sh.at[dst_v.at[j]], add=True)
        return _

    lax.fori_loop(0, CH, chunk, 0)
    plsc.subcore_barrier()
    pltpu.sync_copy(acc_sh.at[pl.ds(base, NPT)],
                    acc_hbm.at[c].at[pl.ds(base, NPT)])


def _ep_body(acc_ref, dis_ref, b_ref, out_ref):
    full = jnp.concatenate([acc_ref[0], acc_ref[1]], axis=-1)
    out_ref[...] = full * dis_ref[...][:, None] + b_ref[...]


def _ep_call(acc, dis, b):
    blk = 1000
    grid = N // blk
    return pl.pallas_call(
        _ep_body,
        grid=(grid,),
        in_specs=[
            pl.BlockSpec((NC, blk, DH), lambda i: (0, i, 0)),
            pl.BlockSpec((blk,), lambda i: (i,)),
            pl.BlockSpec((D,), lambda i: (0,)),
        ],
        out_specs=pl.BlockSpec((blk, D), lambda i: (i, 0)),
        out_shape=jax.ShapeDtypeStruct((N, D), jnp.float32),
    )(acc, dis, b)


def kernel(x, edge_index, W, b):
    src = edge_index[0]
    dst = edge_index[1]
    srcp = jnp.concatenate(
        [src, jnp.zeros((EPAD,), jnp.int32)]).reshape(NS, CH, LANE)
    dstp = jnp.concatenate(
        [dst, jnp.full((EPAD,), DUMMY, jnp.int32)]).reshape(NS, CH, LANE)

    degs = _deg_kernel(dstp)
    deg0 = degs[0, :N]
    deg1 = degs[1, :N]
    hp, dis = _mm_call(x, W, deg0, deg1)
    acc = _scatter_kernel(hp, srcp, dstp)
    return _ep_call(acc, dis, b)


# speedup vs baseline: 17.2165x; 17.2165x over previous
import functools

import jax
import jax.numpy as jnp
from jax import lax
from jax.experimental import pallas as pl
from jax.experimental.pallas import tpu as pltpu
from jax.experimental.pallas import tpu_sc as plsc

N = 10000
E = 320000
D = 128
NC = 2
NS = 16
LANE = 128
CH = 79
EPAD = NC * NS * CH * LANE - E
NPAD = 10240
NPT = NPAD // NS
DUMMY = N
BLK = 1024

_vmesh = plsc.VectorSubcoreMesh(core_axis_name="c", subcore_axis_name="s")


@functools.partial(
    pl.kernel,
    out_type=jax.ShapeDtypeStruct((NC, NPAD), jnp.float32),
    mesh=_vmesh,
    scratch_types=[
        pltpu.VMEM((CH, LANE), jnp.int32),
        pltpu.VMEM((LANE,), jnp.float32),
        pltpu.VMEM((NPT,), jnp.float32),
        pltpu.VMEM_SHARED((NPAD,), jnp.float32),
    ],
)
def _deg_kernel(dst_hbm, deg_hbm, dst_v, ones_v, zeros_v, deg_sh):
    c = lax.axis_index("c")
    s = lax.axis_index("s")

    def fill0(i, _):
        zeros_v[pl.ds(i * 16, 16)] = jnp.zeros((16,), jnp.float32)
        return _

    lax.fori_loop(0, NPT // 16, fill0, 0)

    def fill1(i, _):
        ones_v[pl.ds(i * 16, 16)] = jnp.ones((16,), jnp.float32)
        return _

    lax.fori_loop(0, LANE // 16, fill1, 0)

    base = s * NPT
    pltpu.sync_copy(zeros_v, deg_sh.at[pl.ds(base, NPT)])
    pltpu.sync_copy(dst_hbm.at[c].at[s], dst_v)
    plsc.subcore_barrier()

    def chunk(j, _):
        pltpu.sync_copy(ones_v, deg_sh.at[dst_v.at[j]], add=True)
        return _

    lax.fori_loop(0, CH, chunk, 0)
    plsc.subcore_barrier()
    pltpu.sync_copy(deg_sh.at[pl.ds(base, NPT)],
                    deg_hbm.at[c].at[pl.ds(base, NPT)])


def _mm_body(x_ref, w_ref, deg0_ref, deg1_ref, hp_ref, dis_ref):
    deg = deg0_ref[0, 0] + deg1_ref[0, 0] + 1.0
    dis = lax.rsqrt(deg)
    h = jnp.dot(x_ref[...], w_ref[...], preferred_element_type=jnp.float32)
    hp_ref[...] = h * dis[:, None]
    dis_ref[0, 0] = dis


def _mm_call(x, W, deg0, deg1):
    grid = NPAD // BLK
    return pl.pallas_call(
        _mm_body,
        grid=(grid,),
        in_specs=[
            pl.BlockSpec((BLK, D), lambda i: (i, 0)),
            pl.BlockSpec((D, D), lambda i: (0, 0)),
            pl.BlockSpec((1, 1, BLK), lambda i: (i, 0, 0)),
            pl.BlockSpec((1, 1, BLK), lambda i: (i, 0, 0)),
        ],
        out_specs=[
            pl.BlockSpec((BLK, D), lambda i: (i, 0)),
            pl.BlockSpec((1, 1, BLK), lambda i: (i, 0, 0)),
        ],
        out_shape=[
            jax.ShapeDtypeStruct((NPAD, D), jnp.float32),
            jax.ShapeDtypeStruct((grid, 1, BLK), jnp.float32),
        ],
    )(x, W, deg0, deg1)


@functools.partial(
    pl.kernel,
    out_type=jax.ShapeDtypeStruct((NC, NPAD, D), jnp.float32),
    mesh=_vmesh,
    scratch_types=[
        pltpu.VMEM((CH, LANE), jnp.int32),
        pltpu.VMEM((CH, LANE), jnp.int32),
        pltpu.VMEM((LANE, D), jnp.float32),
        pltpu.VMEM_SHARED((NPAD, D), jnp.float32),
        pltpu.SemaphoreType.DMA,
    ],
)
def _scatter_kernel(hp_hbm, src_hbm, dst_hbm, acc_hbm, src_v, dst_v, rows_v,
                    acc_sh, sem):
    c = lax.axis_index("c")
    s = lax.axis_index("s")
    pltpu.sync_copy(src_hbm.at[c].at[s], src_v)
    pltpu.sync_copy(dst_hbm.at[c].at[s], dst_v)
    base = s * NPT
    pltpu.sync_copy(hp_hbm.at[pl.ds(base, NPT)], acc_sh.at[pl.ds(base, NPT)])
    plsc.subcore_barrier()

    def chunk(j, _):
        pltpu.async_copy(hp_hbm.at[src_v.at[j]], rows_v, sem).wait()
        pltpu.sync_copy(rows_v, acc_sh.at[dst_v.at[j]], add=True)
        return _

    lax.fori_loop(0, CH, chunk, 0)
    plsc.subcore_barrier()
    pltpu.sync_copy(acc_sh.at[pl.ds(base, NPT)],
                    acc_hbm.at[c].at[pl.ds(base, NPT)])


def _ep_body(acc_ref, hp_ref, dis_ref, b_ref, out_ref):
    total = acc_ref[0] + acc_ref[1] - hp_ref[...]
    out_ref[...] = total * dis_ref[0, 0][:, None] + b_ref[...]


def _ep_call(acc, hp, dis, b):
    grid = NPAD // BLK
    return pl.pallas_call(
        _ep_body,
        grid=(grid,),
        in_specs=[
            pl.BlockSpec((NC, BLK, D), lambda i: (0, i, 0)),
            pl.BlockSpec((BLK, D), lambda i: (i, 0)),
            pl.BlockSpec((1, 1, BLK), lambda i: (i, 0, 0)),
            pl.BlockSpec((D,), lambda i: (0,)),
        ],
        out_specs=pl.BlockSpec((BLK, D), lambda i: (i, 0)),
        out_shape=jax.ShapeDtypeStruct((N, D), jnp.float32),
    )(acc, hp, dis, b)


def kernel(x, edge_index, W, b):
    src = edge_index[0]
    dst = edge_index[1]
    srcp = jnp.concatenate(
        [src, jnp.zeros((EPAD,), jnp.int32)]).reshape(NC, NS, CH, LANE)
    dstp = jnp.concatenate(
        [dst, jnp.full((EPAD,), DUMMY, jnp.int32)]).reshape(NC, NS, CH, LANE)

    degs = _deg_kernel(dstp)
    deg0 = degs[0].reshape(NPAD // BLK, 1, BLK)
    deg1 = degs[1].reshape(NPAD // BLK, 1, BLK)
    hp, dis = _mm_call(x, W, deg0, deg1)
    acc = _scatter_kernel(hp, srcp, dstp)
    return _ep_call(acc, hp, dis, b)
